# Initial kernel scaffold; baseline (speedup 1.0000x reference)
import jax
import jax.numpy as jnp
from jax.experimental import pallas as pl


def kernel(x, edge_index, W0, b0, W1, b1, W_out, b_out):
    raise NotImplementedError("write your pallas kernel here")


# speedup vs baseline: 8.6571x; 8.6571x over previous
import functools

import jax
import jax.numpy as jnp
from jax import lax
from jax.experimental import pallas as pl
from jax.experimental.pallas import tpu as pltpu
from jax.experimental.pallas import tpu_sc as plsc

NC = 2
NS = 16
LANES = 16
BATCH = 128
NBATCH = 2
SB = BATCH * NBATCH


def _round_up(a, m):
    return -(-a // m) * m


def _mesh():
    return plsc.VectorSubcoreMesh(
        core_axis_name="c", subcore_axis_name="s", num_cores=NC, num_subcores=NS
    )


@functools.lru_cache(maxsize=None)
def _make_deg_kernel(n_nodes, e_pad):
    half = n_nodes // 2
    d_r = _round_up(half // NS + 1, 32)
    acc_rows = NS * d_r
    sb_per_tile = e_pad // (NS * SB)

    @functools.partial(
        pl.kernel,
        out_type=jax.ShapeDtypeStruct((NC * acc_rows,), jnp.float32),
        mesh=_mesh(),
        compiler_params=pltpu.CompilerParams(use_tc_tiling_on_sc=False),
        scratch_types=[
            pltpu.VMEM_SHARED((acc_rows,), jnp.float32),
            pltpu.VMEM((NBATCH, BATCH), jnp.int32),
            pltpu.VMEM((NBATCH, BATCH), jnp.int32),
            pltpu.VMEM((SB,), jnp.float32),
            pltpu.VMEM((d_r // 4,), jnp.float32),
            pltpu.VMEM((d_r,), jnp.float32),
            pltpu.SemaphoreType.DMA,
        ],
    )
    def deg_kernel(dst_hbm, out_hbm, acc_sh, dst_v, sidx_v, ones_v, zbuf_v, obuf_v, sem):
        cid = lax.axis_index("c")
        sid = lax.axis_index("s")
        lo = cid * half

        zeros16 = jnp.zeros((LANES,), jnp.float32)
        ones16 = jnp.ones((LANES,), jnp.float32)

        def fill(i, _):
            zbuf_v[pl.ds(i * LANES, LANES)] = zeros16
            return 0

        lax.fori_loop(0, d_r // 4 // LANES, fill, 0)

        def fill1(i, _):
            ones_v[pl.ds(i * LANES, LANES)] = ones16
            return 0

        lax.fori_loop(0, SB // LANES, fill1, 0)

        for k in range(4):
            pltpu.sync_copy(
                zbuf_v, acc_sh.at[pl.ds(sid * d_r + k * (d_r // 4), d_r // 4)]
            )
        plsc.subcore_barrier()

        row_base = sid * (sb_per_tile * NBATCH)

        def body(g, _):
            rb = row_base + g * NBATCH
            pltpu.sync_copy(dst_hbm.at[pl.ds(rb, NBATCH)], dst_v)
            for j in range(NBATCH):
                for q in range(BATCH // LANES):
                    d = dst_v[j, pl.ds(q * LANES, LANES)]
                    loc = d - lo
                    ok = (loc >= 0) & (loc < half)
                    sidx_v[j, pl.ds(q * LANES, LANES)] = jnp.where(ok, loc, half)
            cps = []
            for j in range(NBATCH):
                cps.append(
                    pltpu.async_copy(
                        ones_v.at[pl.ds(j * BATCH, BATCH)],
                        acc_sh.at[sidx_v.at[j]],
                        sem,
                        add=True,
                    )
                )
            for c in cps:
                c.wait()
            return 0

        lax.fori_loop(0, sb_per_tile, body, 0)
        plsc.subcore_barrier()
        pltpu.sync_copy(acc_sh.at[pl.ds(sid * d_r, d_r)], obuf_v)
        pltpu.sync_copy(obuf_v, out_hbm.at[pl.ds(cid * acc_rows + sid * d_r, d_r)])

    return deg_kernel, acc_rows


@functools.lru_cache(maxsize=None)
def _make_edge_pass(n_nodes, e_pad, feat):
    half = n_nodes // 2
    d_r = _round_up(half // NS + 1, 32)
    acc_rows = NS * d_r
    sb_per_tile = e_pad // (NS * SB)

    @functools.partial(
        pl.kernel,
        out_type=jax.ShapeDtypeStruct((NC * acc_rows, feat), jnp.float32),
        mesh=_mesh(),
        compiler_params=pltpu.CompilerParams(use_tc_tiling_on_sc=False),
        scratch_types=[
            pltpu.VMEM_SHARED((acc_rows, feat), jnp.float32),
            pltpu.VMEM((NBATCH, BATCH), jnp.int32),
            pltpu.VMEM((NBATCH, BATCH), jnp.int32),
            pltpu.VMEM((NBATCH, BATCH), jnp.int32),
            pltpu.VMEM((SB, feat), jnp.float32),
            pltpu.SemaphoreType.DMA,
            pltpu.SemaphoreType.DMA,
        ],
    )
    def edge_pass(
        g_hbm, src_hbm, dst_hbm, out_hbm,
        acc_sh, src_v, dst_v, sidx_v, rows_v, sem_g, sem_s,
    ):
        cid = lax.axis_index("c")
        sid = lax.axis_index("s")
        lo = cid * half

        zeros16 = jnp.zeros((LANES,), jnp.float32)

        def fill(i, _):
            for k in range(feat // LANES):
                rows_v[i, pl.ds(k * LANES, LANES)] = zeros16
            return 0

        lax.fori_loop(0, SB, fill, 0)
        zc = max(c for c in range(1, SB + 1) if d_r % c == 0)
        for k in range(d_r // zc):
            pltpu.sync_copy(
                rows_v.at[pl.ds(0, zc)],
                acc_sh.at[pl.ds(sid * d_r + k * zc, zc)],
            )
        plsc.subcore_barrier()

        row_base = sid * (sb_per_tile * NBATCH)

        def body(g, _):
            rb = row_base + g * NBATCH
            pltpu.sync_copy(src_hbm.at[pl.ds(rb, NBATCH)], src_v)
            pltpu.sync_copy(dst_hbm.at[pl.ds(rb, NBATCH)], dst_v)
            for j in range(NBATCH):
                for q in range(BATCH // LANES):
                    d = dst_v[j, pl.ds(q * LANES, LANES)]
                    loc = d - lo
                    ok = (loc >= 0) & (loc < half)
                    sidx_v[j, pl.ds(q * LANES, LANES)] = jnp.where(ok, loc, half)
            gcps = []
            for j in range(NBATCH):
                gcps.append(
                    pltpu.async_copy(
                        g_hbm.at[src_v.at[j]],
                        rows_v.at[pl.ds(j * BATCH, BATCH)],
                        sem_g,
                    )
                )
            for c in gcps:
                c.wait()
            scps = []
            for j in range(NBATCH):
                scps.append(
                    pltpu.async_copy(
                        rows_v.at[pl.ds(j * BATCH, BATCH)],
                        acc_sh.at[sidx_v.at[j]],
                        sem_s,
                        add=True,
                    )
                )
            for c in scps:
                c.wait()
            return 0

        lax.fori_loop(0, sb_per_tile, body, 0)
        plsc.subcore_barrier()
        chunk = d_r // 4
        for k in range(4):
            pltpu.sync_copy(
                acc_sh.at[pl.ds(sid * d_r + k * chunk, chunk)],
                rows_v.at[pl.ds(0, chunk)],
            )
            pltpu.sync_copy(
                rows_v.at[pl.ds(0, chunk)],
                out_hbm.at[pl.ds(cid * acc_rows + sid * d_r + k * chunk, chunk)],
            )

    return edge_pass, acc_rows


_TC_R = 2000


def _row_spec(r, cols):
    return pl.BlockSpec((r, cols), lambda i: (i, 0))


def _full_spec(rows, cols):
    return pl.BlockSpec((rows, cols), lambda i: (0, 0))


def _ka_body(deg_ref, x_ref, w_ref, g_ref, dinv_ref):
    dinv = lax.rsqrt(deg_ref[...] + 1.0)
    g_ref[...] = (
        jnp.dot(x_ref[...], w_ref[...], preferred_element_type=jnp.float32) * dinv
    )
    dinv_ref[...] = dinv


def _kb_body(acc_ref, g0_ref, dinv_ref, b_ref, w_ref, h1_ref, g1_ref):
    dinv = dinv_ref[...]
    h1 = jnp.maximum(dinv * (acc_ref[...] + g0_ref[...]) + b_ref[...], 0.0)
    h1_ref[...] = h1
    g1_ref[...] = (
        jnp.dot(h1, w_ref[...], preferred_element_type=jnp.float32) * dinv
    )


def _kc_body(acc_ref, g1_ref, h1_ref, dinv_ref, b_ref, w_ref, g2a_ref, g2b_ref):
    dinv = dinv_ref[...]
    h2 = (
        jnp.maximum(dinv * (acc_ref[...] + g1_ref[...]) + b_ref[...], 0.0)
        + h1_ref[...]
    )
    p = jnp.dot(h2, w_ref[...], preferred_element_type=jnp.float32) * dinv
    g2a_ref[...] = p[:, :32]
    g2b_ref[...] = p[:, 32:]


def _kd_body(acca_ref, accb_ref, ga_ref, gb_ref, dinv_ref, b_ref, out_ref):
    dinv = dinv_ref[...]
    b = b_ref[...]
    out_ref[:, :32] = dinv * (acca_ref[...] + ga_ref[...]) + b[:, :32]
    out_ref[:, 32:] = dinv * (accb_ref[...] + gb_ref[...]) + b[:, 32:]


def kernel(x, edge_index, W0, b0, W1, b1, W_out, b_out):
    n, dfeat = x.shape
    e = edge_index.shape[1]
    nh = W0.shape[1]
    nclass = W_out.shape[1]
    half = n // 2
    r = _TC_R if n % _TC_R == 0 else n // 8
    grid = (n // r,)

    src = edge_index[0]
    dst = edge_index[1]
    e_pad = _round_up(e, NS * SB)
    pad = e_pad - e
    src_p = jnp.concatenate(
        [src, jnp.zeros((pad,), jnp.int32)]
    ).reshape(-1, BATCH)
    dst_p = jnp.concatenate(
        [dst, jnp.full((pad,), jnp.int32(n))]
    ).reshape(-1, BATCH)

    deg_kernel, acc_rows = _make_deg_kernel(n, e_pad)
    deg_padded = deg_kernel(dst_p)
    deg = jnp.concatenate(
        [deg_padded[:half], deg_padded[acc_rows : acc_rows + half]]
    ).reshape(n, 1)

    edge_pass_fn, ep_rows = _make_edge_pass(n, e_pad, nh)

    def edge_pass(g):
        acc_p = edge_pass_fn(g, src_p, dst_p)
        return jnp.concatenate(
            [acc_p[:half], acc_p[ep_rows : ep_rows + half]]
        )

    ka = pl.pallas_call(
        _ka_body,
        grid=grid,
        in_specs=[
            _row_spec(r, 1),
            _row_spec(r, dfeat),
            _full_spec(dfeat, nh),
        ],
        out_specs=[_row_spec(r, nh), _row_spec(r, 1)],
        out_shape=[
            jax.ShapeDtypeStruct((n, nh), jnp.float32),
            jax.ShapeDtypeStruct((n, 1), jnp.float32),
        ],
    )
    g0, dinv = ka(deg, x, W0)

    acc0 = edge_pass(g0)

    kb = pl.pallas_call(
        _kb_body,
        grid=grid,
        in_specs=[
            _row_spec(r, nh),
            _row_spec(r, nh),
            _row_spec(r, 1),
            _full_spec(1, nh),
            _full_spec(nh, nh),
        ],
        out_specs=[_row_spec(r, nh), _row_spec(r, nh)],
        out_shape=[
            jax.ShapeDtypeStruct((n, nh), jnp.float32),
            jax.ShapeDtypeStruct((n, nh), jnp.float32),
        ],
    )
    h1, g1 = kb(acc0, g0, dinv, b0.reshape(1, nh), W1)

    acc1 = edge_pass(g1)

    kc = pl.pallas_call(
        _kc_body,
        grid=grid,
        in_specs=[
            _row_spec(r, nh),
            _row_spec(r, nh),
            _row_spec(r, nh),
            _row_spec(r, 1),
            _full_spec(1, nh),
            _full_spec(nh, nclass),
        ],
        out_specs=[_row_spec(r, nh), _row_spec(r, nh)],
        out_shape=[
            jax.ShapeDtypeStruct((n, nh), jnp.float32),
            jax.ShapeDtypeStruct((n, nh), jnp.float32),
        ],
    )
    g2a, g2b = kc(acc1, g1, h1, dinv, b1.reshape(1, nh), W_out)

    acc2a = edge_pass(g2a)
    acc2b = edge_pass(g2b)

    kd = pl.pallas_call(
        _kd_body,
        grid=grid,
        in_specs=[
            _row_spec(r, nh),
            _row_spec(r, nh),
            _row_spec(r, nh),
            _row_spec(r, nh),
            _row_spec(r, 1),
            _full_spec(1, nclass),
        ],
        out_specs=_row_spec(r, nclass),
        out_shape=jax.ShapeDtypeStruct((n, nclass), jnp.float32),
    )
    out = kd(acc2a, acc2b, g2a, g2b, dinv, b_out.reshape(1, nclass))
    return out


# speedup vs baseline: 20.4738x; 2.3650x over previous
import functools

import jax
import jax.numpy as jnp
from jax import lax
from jax.experimental import pallas as pl
from jax.experimental.pallas import tpu as pltpu
from jax.experimental.pallas import tpu_sc as plsc

NC = 2
NS = 16
LANES = 16
BATCH = 128
NBATCH = 4
SB = BATCH * NBATCH
FH = 16


def _round_up(a, m):
    return -(-a // m) * m


def _chunk_of(total, cap, align=1):
    return max(c for c in range(1, cap + 1)
               if total % c == 0 and c % align == 0)


def _mesh():
    return plsc.VectorSubcoreMesh(
        core_axis_name="c", subcore_axis_name="s", num_cores=NC, num_subcores=NS
    )


_SC_PARAMS = pltpu.CompilerParams(use_tc_tiling_on_sc=False)


@functools.lru_cache(maxsize=None)
def _make_deg_kernel(n_nodes, e_pad):
    d_r = _round_up(n_nodes // NS + 1, 8)
    acc_rows = NS * d_r
    sb_per_tile = e_pad // (NC * NS * SB)
    zc = _chunk_of(d_r, SB, align=8)

    @functools.partial(
        pl.kernel,
        out_type=jax.ShapeDtypeStruct((NC * acc_rows,), jnp.float32),
        mesh=_mesh(),
        compiler_params=_SC_PARAMS,
        scratch_types=[
            pltpu.VMEM_SHARED((acc_rows,), jnp.float32),
            pltpu.VMEM((NBATCH, BATCH), jnp.int32),
            pltpu.VMEM((NBATCH, BATCH), jnp.int32),
            pltpu.VMEM((SB,), jnp.float32),
            pltpu.VMEM((d_r,), jnp.float32),
            pltpu.SemaphoreType.DMA,
            pltpu.SemaphoreType.DMA,
        ],
    )
    def deg_kernel(dst_hbm, out_hbm, acc_sh, dstA, dstB, ones_v, obuf_v,
                   sem_i, sem_s):
        cid = lax.axis_index("c")
        sid = lax.axis_index("s")

        zeros16 = jnp.zeros((LANES,), jnp.float32)
        ones16 = jnp.ones((LANES,), jnp.float32)

        def fill0(i, _):
            obuf_v[pl.ds(i * LANES, LANES)] = zeros16
            return 0

        lax.fori_loop(0, d_r // LANES, fill0, 0)

        def fill1(i, _):
            ones_v[pl.ds(i * LANES, LANES)] = ones16
            return 0

        lax.fori_loop(0, SB // LANES, fill1, 0)

        for k in range(d_r // zc):
            pltpu.sync_copy(
                obuf_v.at[pl.ds(0, zc)],
                acc_sh.at[pl.ds(sid * d_r + k * zc, zc)],
            )
        plsc.subcore_barrier()

        row_base = (cid * NS + sid) * (sb_per_tile * NBATCH)
        nsb = sb_per_tile

        def fire_scatters(dst_v):
            for j in range(NBATCH):
                pltpu.async_copy(
                    ones_v.at[pl.ds(j * BATCH, BATCH)],
                    acc_sh.at[dst_v.at[j]],
                    sem_s,
                    add=True,
                )

        def wait_scatters(dst_v):
            for j in range(NBATCH):
                pltpu.make_async_copy(
                    ones_v.at[pl.ds(j * BATCH, BATCH)],
                    acc_sh.at[dst_v.at[j]],
                    sem_s,
                ).wait()

        pltpu.sync_copy(dst_hbm.at[pl.ds(row_base, NBATCH)], dstA)

        def one_iter(g, cur, prev):
            @pl.when(g > 0)
            def _():
                pltpu.make_async_copy(
                    dst_hbm.at[pl.ds(row_base, NBATCH)], cur, sem_i
                ).wait()

            fire_scatters(cur)

            @pl.when(g > 0)
            def _():
                wait_scatters(prev)

            @pl.when(g + 1 < nsb)
            def _():
                pltpu.async_copy(
                    dst_hbm.at[pl.ds(row_base + (g + 1) * NBATCH, NBATCH)],
                    prev,
                    sem_i,
                )

        def body(g, _):
            @pl.when(g % 2 == 0)
            def _():
                one_iter(g, dstA, dstB)

            @pl.when(g % 2 == 1)
            def _():
                one_iter(g, dstB, dstA)

            return 0

        lax.fori_loop(0, nsb, body, 0)
        wait_scatters(dstA if (nsb - 1) % 2 == 0 else dstB)
        plsc.subcore_barrier()

        pltpu.sync_copy(acc_sh.at[pl.ds(sid * d_r, d_r)], obuf_v)
        pltpu.sync_copy(obuf_v, out_hbm.at[pl.ds(cid * acc_rows + sid * d_r, d_r)])

    return deg_kernel, acc_rows


@functools.lru_cache(maxsize=None)
def _make_edge_pass(n_nodes, e_pad):
    d_r = _round_up(n_nodes // NS + 1, 8)
    acc_rows = NS * d_r
    out_rows = n_nodes // NS
    sb_per_tile = e_pad // (NS * SB)
    zc = _chunk_of(d_r, SB)
    oc = _chunk_of(out_rows, SB)

    @functools.partial(
        pl.kernel,
        out_type=(
            jax.ShapeDtypeStruct((n_nodes, FH), jnp.float32),
            jax.ShapeDtypeStruct((n_nodes, FH), jnp.float32),
        ),
        mesh=_mesh(),
        compiler_params=_SC_PARAMS,
        scratch_types=[
            pltpu.VMEM_SHARED((acc_rows, FH), jnp.float32),
            pltpu.VMEM((NBATCH, BATCH), jnp.int32),
            pltpu.VMEM((NBATCH, BATCH), jnp.int32),
            pltpu.VMEM((NBATCH, BATCH), jnp.int32),
            pltpu.VMEM((NBATCH, BATCH), jnp.int32),
            pltpu.VMEM((SB, FH), jnp.float32),
            pltpu.VMEM((SB, FH), jnp.float32),
            pltpu.SemaphoreType.DMA,
            pltpu.SemaphoreType.DMA,
            pltpu.SemaphoreType.DMA,
            pltpu.SemaphoreType.DMA,
        ],
    )
    def edge_pass(
        g_lo, g_hi, src_hbm, dst_hbm, out_lo, out_hi,
        acc_sh, srcA, dstA, srcB, dstB, rowsA, rowsB,
        sem_i, sem_g, sem_s, sem_o,
    ):
        cid = lax.axis_index("c")
        sid = lax.axis_index("s")

        zeros16 = jnp.zeros((LANES,), jnp.float32)

        def fill0(i, _):
            rowsA[i, pl.ds(0, LANES)] = zeros16
            return 0

        lax.fori_loop(0, SB, fill0, 0)
        for k in range(d_r // zc):
            pltpu.sync_copy(
                rowsA.at[pl.ds(0, zc)],
                acc_sh.at[pl.ds(sid * d_r + k * zc, zc)],
            )
        plsc.subcore_barrier()

        row_base = sid * (sb_per_tile * NBATCH)
        nsb = sb_per_tile

        def _run(table, out_hbm):
            def fire_gathers(src_v, rows_v):
                for j in range(NBATCH):
                    pltpu.async_copy(
                        table.at[src_v.at[j]],
                        rows_v.at[pl.ds(j * BATCH, BATCH)],
                        sem_g,
                    )

            def wait_gathers(src_v, rows_v):
                for j in range(NBATCH):
                    pltpu.make_async_copy(
                        table.at[src_v.at[j]],
                        rows_v.at[pl.ds(j * BATCH, BATCH)],
                        sem_g,
                    ).wait()

            def fire_scatters(dst_v, rows_v):
                for j in range(NBATCH):
                    pltpu.async_copy(
                        rows_v.at[pl.ds(j * BATCH, BATCH)],
                        acc_sh.at[dst_v.at[j]],
                        sem_s,
                        add=True,
                    )

            def wait_scatters(dst_v, rows_v):
                for j in range(NBATCH):
                    pltpu.make_async_copy(
                        rows_v.at[pl.ds(j * BATCH, BATCH)],
                        acc_sh.at[dst_v.at[j]],
                        sem_s,
                    ).wait()

            pltpu.sync_copy(src_hbm.at[pl.ds(row_base, NBATCH)], srcA)
            pltpu.sync_copy(dst_hbm.at[pl.ds(row_base, NBATCH)], dstA)

            def one_iter(g, cur_src, cur_dst, cur_rows, prv_src, prv_dst,
                         prv_rows):
                @pl.when(g > 0)
                def _():
                    pltpu.make_async_copy(
                        src_hbm.at[pl.ds(row_base, NBATCH)], cur_src, sem_i
                    ).wait()
                    pltpu.make_async_copy(
                        dst_hbm.at[pl.ds(row_base, NBATCH)], cur_dst, sem_i
                    ).wait()

                fire_gathers(cur_src, cur_rows)

                @pl.when(g > 0)
                def _():
                    wait_gathers(prv_src, prv_rows)
                    fire_scatters(prv_dst, prv_rows)
                    wait_scatters(prv_dst, prv_rows)

                @pl.when(g + 1 < nsb)
                def _():
                    rb1 = row_base + (g + 1) * NBATCH
                    pltpu.async_copy(src_hbm.at[pl.ds(rb1, NBATCH)], prv_src,
                                     sem_i)
                    pltpu.async_copy(dst_hbm.at[pl.ds(rb1, NBATCH)], prv_dst,
                                     sem_i)

            def body(g, _):
                @pl.when(g % 2 == 0)
                def _():
                    one_iter(g, srcA, dstA, rowsA, srcB, dstB, rowsB)

                @pl.when(g % 2 == 1)
                def _():
                    one_iter(g, srcB, dstB, rowsB, srcA, dstA, rowsA)

                return 0

            lax.fori_loop(0, nsb, body, 0)
            if (nsb - 1) % 2 == 0:
                lsrc, ldst, lrows = srcA, dstA, rowsA
            else:
                lsrc, ldst, lrows = srcB, dstB, rowsB
            wait_gathers(lsrc, lrows)
            fire_scatters(ldst, lrows)
            wait_scatters(ldst, lrows)
            plsc.subcore_barrier()

            nchunks = out_rows // oc
            for k in range(nchunks):
                rbuf = rowsA if k % 2 == 0 else rowsB
                if k >= 2:
                    pltpu.make_async_copy(
                        rbuf.at[pl.ds(0, oc)],
                        out_hbm.at[pl.ds(sid * out_rows, oc)],
                        sem_o,
                    ).wait()
                pltpu.sync_copy(
                    acc_sh.at[pl.ds(sid * out_rows + k * oc, oc)],
                    rbuf.at[pl.ds(0, oc)],
                )
                pltpu.async_copy(
                    rbuf.at[pl.ds(0, oc)],
                    out_hbm.at[pl.ds(sid * out_rows + k * oc, oc)],
                    sem_o,
                )
            for k in range(min(2, nchunks)):
                pltpu.make_async_copy(
                    rowsA.at[pl.ds(0, oc)],
                    out_hbm.at[pl.ds(sid * out_rows, oc)],
                    sem_o,
                ).wait()

        @pl.when(cid == 0)
        def _():
            _run(g_lo, out_lo)

        @pl.when(cid == 1)
        def _():
            _run(g_hi, out_hi)

    return edge_pass


_TC_R = 2000


def _row_spec(r, cols):
    return pl.BlockSpec((r, cols), lambda i: (i, 0))


def _full_spec(rows, cols):
    return pl.BlockSpec((rows, cols), lambda i: (0, 0))


def _ka_body(deg0_ref, deg1_ref, x_ref, w_ref, glo_ref, ghi_ref, dinv_ref):
    dinv = lax.rsqrt(deg0_ref[...] + deg1_ref[...] + 1.0)
    g = jnp.dot(x_ref[...], w_ref[...], preferred_element_type=jnp.float32) * dinv
    glo_ref[...] = g[:, :FH]
    ghi_ref[...] = g[:, FH:]
    dinv_ref[...] = dinv


def _kb_body(alo_ref, ahi_ref, glo_ref, ghi_ref, dinv_ref, b_ref, w_ref,
             h1_ref, g1lo_ref, g1hi_ref):
    dinv = dinv_ref[...]
    acc = jnp.concatenate([alo_ref[...], ahi_ref[...]], axis=1)
    g0 = jnp.concatenate([glo_ref[...], ghi_ref[...]], axis=1)
    h1 = jnp.maximum(dinv * (acc + g0) + b_ref[...], 0.0)
    h1_ref[...] = h1
    g1 = jnp.dot(h1, w_ref[...], preferred_element_type=jnp.float32) * dinv
    g1lo_ref[...] = g1[:, :FH]
    g1hi_ref[...] = g1[:, FH:]


def _kc_body(alo_ref, ahi_ref, glo_ref, ghi_ref, h1_ref, dinv_ref, b_ref,
             w_ref, q0_ref, q1_ref, q2_ref, q3_ref):
    dinv = dinv_ref[...]
    acc = jnp.concatenate([alo_ref[...], ahi_ref[...]], axis=1)
    g1 = jnp.concatenate([glo_ref[...], ghi_ref[...]], axis=1)
    h2 = jnp.maximum(dinv * (acc + g1) + b_ref[...], 0.0) + h1_ref[...]
    p = jnp.dot(h2, w_ref[...], preferred_element_type=jnp.float32) * dinv
    q0_ref[...] = p[:, 0 * FH : 1 * FH]
    q1_ref[...] = p[:, 1 * FH : 2 * FH]
    q2_ref[...] = p[:, 2 * FH : 3 * FH]
    q3_ref[...] = p[:, 3 * FH : 4 * FH]


def _kd_body(a0_ref, a1_ref, a2_ref, a3_ref, q0_ref, q1_ref, q2_ref, q3_ref,
             dinv_ref, b_ref, out_ref):
    dinv = dinv_ref[...]
    b = b_ref[...]
    for i, (a, q) in enumerate(
        ((a0_ref, q0_ref), (a1_ref, q1_ref), (a2_ref, q2_ref), (a3_ref, q3_ref))
    ):
        out_ref[:, i * FH : (i + 1) * FH] = (
            dinv * (a[...] + q[...]) + b[:, i * FH : (i + 1) * FH]
        )


def kernel(x, edge_index, W0, b0, W1, b1, W_out, b_out):
    n, dfeat = x.shape
    e = edge_index.shape[1]
    nh = W0.shape[1]
    nclass = W_out.shape[1]
    r = _TC_R if n % _TC_R == 0 else n // 8
    grid = (n // r,)

    src = edge_index[0]
    dst = edge_index[1]
    e_pad = _round_up(e, NC * NS * SB)
    pad = e_pad - e
    src_p = jnp.concatenate([src, jnp.zeros((pad,), jnp.int32)]).reshape(-1, BATCH)
    dst_p = jnp.concatenate([dst, jnp.full((pad,), jnp.int32(n))]).reshape(-1, BATCH)

    deg_kernel, deg_rows = _make_deg_kernel(n, e_pad)
    deg_pp = deg_kernel(dst_p)
    deg0 = deg_pp[:n].reshape(n, 1)
    deg1 = deg_pp[deg_rows : deg_rows + n].reshape(n, 1)

    edge_pass = _make_edge_pass(n, e_pad)

    ka = pl.pallas_call(
        _ka_body,
        grid=grid,
        in_specs=[
            _row_spec(r, 1),
            _row_spec(r, 1),
            _row_spec(r, dfeat),
            _full_spec(dfeat, nh),
        ],
        out_specs=[_row_spec(r, FH), _row_spec(r, FH), _row_spec(r, 1)],
        out_shape=[
            jax.ShapeDtypeStruct((n, FH), jnp.float32),
            jax.ShapeDtypeStruct((n, FH), jnp.float32),
            jax.ShapeDtypeStruct((n, 1), jnp.float32),
        ],
    )
    g0lo, g0hi, dinv = ka(deg0, deg1, x, W0)

    a0lo, a0hi = edge_pass(g0lo, g0hi, src_p, dst_p)

    kb = pl.pallas_call(
        _kb_body,
        grid=grid,
        in_specs=[
            _row_spec(r, FH),
            _row_spec(r, FH),
            _row_spec(r, FH),
            _row_spec(r, FH),
            _row_spec(r, 1),
            _full_spec(1, nh),
            _full_spec(nh, nh),
        ],
        out_specs=[_row_spec(r, nh), _row_spec(r, FH), _row_spec(r, FH)],
        out_shape=[
            jax.ShapeDtypeStruct((n, nh), jnp.float32),
            jax.ShapeDtypeStruct((n, FH), jnp.float32),
            jax.ShapeDtypeStruct((n, FH), jnp.float32),
        ],
    )
    h1, g1lo, g1hi = kb(a0lo, a0hi, g0lo, g0hi, dinv, b0.reshape(1, nh), W1)

    a1lo, a1hi = edge_pass(g1lo, g1hi, src_p, dst_p)

    kc = pl.pallas_call(
        _kc_body,
        grid=grid,
        in_specs=[
            _row_spec(r, FH),
            _row_spec(r, FH),
            _row_spec(r, FH),
            _row_spec(r, FH),
            _row_spec(r, nh),
            _row_spec(r, 1),
            _full_spec(1, nh),
            _full_spec(nh, nclass),
        ],
        out_specs=[
            _row_spec(r, FH),
            _row_spec(r, FH),
            _row_spec(r, FH),
            _row_spec(r, FH),
        ],
        out_shape=[
            jax.ShapeDtypeStruct((n, FH), jnp.float32),
            jax.ShapeDtypeStruct((n, FH), jnp.float32),
            jax.ShapeDtypeStruct((n, FH), jnp.float32),
            jax.ShapeDtypeStruct((n, FH), jnp.float32),
        ],
    )
    q0, q1, q2, q3 = kc(a1lo, a1hi, g1lo, g1hi, h1, dinv, b1.reshape(1, nh),
                        W_out)

    a2q0, a2q1 = edge_pass(q0, q1, src_p, dst_p)
    a2q2, a2q3 = edge_pass(q2, q3, src_p, dst_p)

    kd = pl.pallas_call(
        _kd_body,
        grid=grid,
        in_specs=[
            _row_spec(r, FH),
            _row_spec(r, FH),
            _row_spec(r, FH),
            _row_spec(r, FH),
            _row_spec(r, FH),
            _row_spec(r, FH),
            _row_spec(r, FH),
            _row_spec(r, FH),
            _row_spec(r, 1),
            _full_spec(1, nclass),
        ],
        out_specs=_row_spec(r, nclass),
        out_shape=jax.ShapeDtypeStruct((n, nclass), jnp.float32),
    )
    out = kd(a2q0, a2q1, a2q2, a2q3, q0, q1, q2, q3, dinv,
             b_out.reshape(1, nclass))
    return out


# speedup vs baseline: 23.8720x; 1.1660x over previous
import functools

import numpy as np
import jax
import jax.numpy as jnp
from jax import lax
from jax.experimental import pallas as pl
from jax.experimental.pallas import tpu as pltpu
from jax.experimental.pallas import tpu_sc as plsc

NC = 2
NS = 16
LANES = 16
BATCH = 128
NBATCH = 4
SB = BATCH * NBATCH
FH = 16
_TC_R = 2048


def _round_up(a, m):
    return -(-a // m) * m


def _chunk_of(total, cap, align=1):
    return max(c for c in range(1, cap + 1)
               if total % c == 0 and c % align == 0)


def _mesh():
    return plsc.VectorSubcoreMesh(
        core_axis_name="c", subcore_axis_name="s", num_cores=NC, num_subcores=NS
    )


_SC_PARAMS = pltpu.CompilerParams(use_tc_tiling_on_sc=False)


@functools.lru_cache(maxsize=None)
def _make_deg_kernel(n_sc, e_pad):
    d_r = n_sc // NS
    acc_rows = n_sc
    sb_per_tile = e_pad // (NC * NS * SB)
    zc = _chunk_of(d_r, SB, align=8)

    @functools.partial(
        pl.kernel,
        out_type=jax.ShapeDtypeStruct((NC * acc_rows,), jnp.float32),
        mesh=_mesh(),
        compiler_params=_SC_PARAMS,
        scratch_types=[
            pltpu.VMEM_SHARED((acc_rows,), jnp.float32),
            pltpu.VMEM((NBATCH, BATCH), jnp.int32),
            pltpu.VMEM((NBATCH, BATCH), jnp.int32),
            pltpu.VMEM((SB,), jnp.float32),
            pltpu.VMEM((d_r,), jnp.float32),
            pltpu.SemaphoreType.DMA,
            pltpu.SemaphoreType.DMA,
        ],
    )
    def deg_kernel(dst_hbm, out_hbm, acc_sh, dstA, dstB, ones_v, obuf_v,
                   sem_i, sem_s):
        cid = lax.axis_index("c")
        sid = lax.axis_index("s")

        zeros16 = jnp.zeros((LANES,), jnp.float32)
        ones16 = jnp.ones((LANES,), jnp.float32)

        def fill0(i, _):
            obuf_v[pl.ds(i * LANES, LANES)] = zeros16
            return 0

        lax.fori_loop(0, d_r // LANES, fill0, 0)

        def fill1(i, _):
            ones_v[pl.ds(i * LANES, LANES)] = ones16
            return 0

        lax.fori_loop(0, SB // LANES, fill1, 0)

        for k in range(d_r // zc):
            pltpu.sync_copy(
                obuf_v.at[pl.ds(0, zc)],
                acc_sh.at[pl.ds(sid * d_r + k * zc, zc)],
            )
        plsc.subcore_barrier()

        row_base = (cid * NS + sid) * (sb_per_tile * NBATCH)
        nsb = sb_per_tile

        def fire_scatters(dst_v):
            for j in range(NBATCH):
                pltpu.async_copy(
                    ones_v.at[pl.ds(j * BATCH, BATCH)],
                    acc_sh.at[dst_v.at[j]],
                    sem_s,
                    add=True,
                )

        def wait_scatters(dst_v):
            for j in range(NBATCH):
                pltpu.make_async_copy(
                    ones_v.at[pl.ds(j * BATCH, BATCH)],
                    acc_sh.at[dst_v.at[j]],
                    sem_s,
                ).wait()

        pltpu.sync_copy(dst_hbm.at[pl.ds(row_base, NBATCH)], dstA)

        def one_iter(g, cur, prev):
            @pl.when(g > 0)
            def _():
                pltpu.make_async_copy(
                    dst_hbm.at[pl.ds(row_base, NBATCH)], cur, sem_i
                ).wait()

            fire_scatters(cur)

            @pl.when(g > 0)
            def _():
                wait_scatters(prev)

            @pl.when(g + 1 < nsb)
            def _():
                pltpu.async_copy(
                    dst_hbm.at[pl.ds(row_base + (g + 1) * NBATCH, NBATCH)],
                    prev,
                    sem_i,
                )

        def body(g, _):
            @pl.when(g % 2 == 0)
            def _():
                one_iter(g, dstA, dstB)

            @pl.when(g % 2 == 1)
            def _():
                one_iter(g, dstB, dstA)

            return 0

        lax.fori_loop(0, nsb, body, 0)
        wait_scatters(dstA if (nsb - 1) % 2 == 0 else dstB)
        plsc.subcore_barrier()

        pltpu.sync_copy(acc_sh.at[pl.ds(sid * d_r, d_r)], obuf_v)
        pltpu.sync_copy(obuf_v, out_hbm.at[pl.ds(cid * acc_rows + sid * d_r, d_r)])

    return deg_kernel


@functools.lru_cache(maxsize=None)
def _make_edge_pass(n_sc, e_pad, k_int, off):
    d_r = n_sc // NS
    out_rows = n_sc // NS
    sb_per_tile = e_pad // (NS * SB)
    zc = _chunk_of(d_r, SB, align=8)
    oc = _chunk_of(out_rows, SB, align=8)

    @functools.partial(
        pl.kernel,
        out_type=(
            jax.ShapeDtypeStruct((n_sc, FH), jnp.float32),
            jax.ShapeDtypeStruct((n_sc, FH), jnp.float32),
        ),
        mesh=_mesh(),
        compiler_params=_SC_PARAMS,
        scratch_types=[
            pltpu.VMEM_SHARED((n_sc, FH), jnp.float32),
            pltpu.VMEM((NBATCH, BATCH), jnp.int32),
            pltpu.VMEM((NBATCH, BATCH), jnp.int32),
            pltpu.VMEM((NBATCH, BATCH), jnp.int32),
            pltpu.VMEM((NBATCH, BATCH), jnp.int32),
            pltpu.VMEM((NBATCH, BATCH), jnp.int32),
            pltpu.VMEM((NBATCH, BATCH), jnp.int32),
            pltpu.VMEM((SB, FH), jnp.float32),
            pltpu.VMEM((SB, FH), jnp.float32),
            pltpu.SemaphoreType.DMA,
            pltpu.SemaphoreType.DMA,
            pltpu.SemaphoreType.DMA,
            pltpu.SemaphoreType.DMA,
        ],
    )
    def edge_pass(
        table, src_hbm, dst_hbm, out_lo, out_hi,
        acc_sh, srcA, dstA, gidxA, srcB, dstB, gidxB, rowsA, rowsB,
        sem_i, sem_g, sem_s, sem_o,
    ):
        cid = lax.axis_index("c")
        sid = lax.axis_index("s")
        qoff = off + cid

        zeros16 = jnp.zeros((LANES,), jnp.float32)

        def fill0(i, _):
            rowsA[i, pl.ds(0, LANES)] = zeros16
            return 0

        lax.fori_loop(0, SB, fill0, 0)
        for k in range(d_r // zc):
            pltpu.sync_copy(
                rowsA.at[pl.ds(0, zc)],
                acc_sh.at[pl.ds(sid * d_r + k * zc, zc)],
            )
        plsc.subcore_barrier()

        row_base = sid * (sb_per_tile * NBATCH)
        nsb = sb_per_tile

        def compute_gidx(src_v, gidx_v):
            for j in range(NBATCH):
                for q in range(BATCH // LANES):
                    s16 = src_v[j, pl.ds(q * LANES, LANES)]
                    gidx_v[j, pl.ds(q * LANES, LANES)] = s16 * k_int + qoff

        def fire_gathers(gidx_v, rows_v):
            for j in range(NBATCH):
                pltpu.async_copy(
                    table.at[gidx_v.at[j]],
                    rows_v.at[pl.ds(j * BATCH, BATCH)],
                    sem_g,
                )

        def wait_gathers(gidx_v, rows_v):
            for j in range(NBATCH):
                pltpu.make_async_copy(
                    table.at[gidx_v.at[j]],
                    rows_v.at[pl.ds(j * BATCH, BATCH)],
                    sem_g,
                ).wait()

        def fire_scatters(dst_v, rows_v):
            for j in range(NBATCH):
                pltpu.async_copy(
                    rows_v.at[pl.ds(j * BATCH, BATCH)],
                    acc_sh.at[dst_v.at[j]],
                    sem_s,
                    add=True,
                )

        def wait_scatters(dst_v, rows_v):
            for j in range(NBATCH):
                pltpu.make_async_copy(
                    rows_v.at[pl.ds(j * BATCH, BATCH)],
                    acc_sh.at[dst_v.at[j]],
                    sem_s,
                ).wait()

        pltpu.sync_copy(src_hbm.at[pl.ds(row_base, NBATCH)], srcA)
        pltpu.sync_copy(dst_hbm.at[pl.ds(row_base, NBATCH)], dstA)
        compute_gidx(srcA, gidxA)

        def one_iter(g, cur_gidx, cur_src, cur_dst, cur_rows,
                     prv_gidx, prv_src, prv_dst, prv_rows):
            @pl.when(g > 0)
            def _():
                pltpu.make_async_copy(
                    src_hbm.at[pl.ds(row_base, NBATCH)], cur_src, sem_i
                ).wait()
                pltpu.make_async_copy(
                    dst_hbm.at[pl.ds(row_base, NBATCH)], cur_dst, sem_i
                ).wait()
                compute_gidx(cur_src, cur_gidx)

            fire_gathers(cur_gidx, cur_rows)

            @pl.when(g > 0)
            def _():
                wait_gathers(prv_gidx, prv_rows)
                fire_scatters(prv_dst, prv_rows)
                wait_scatters(prv_dst, prv_rows)

            @pl.when(g + 1 < nsb)
            def _():
                rb1 = row_base + (g + 1) * NBATCH
                pltpu.async_copy(src_hbm.at[pl.ds(rb1, NBATCH)], prv_src, sem_i)
                pltpu.async_copy(dst_hbm.at[pl.ds(rb1, NBATCH)], prv_dst, sem_i)

        def body(g, _):
            @pl.when(g % 2 == 0)
            def _():
                one_iter(g, gidxA, srcA, dstA, rowsA, gidxB, srcB, dstB, rowsB)

            @pl.when(g % 2 == 1)
            def _():
                one_iter(g, gidxB, srcB, dstB, rowsB, gidxA, srcA, dstA, rowsA)

            return 0

        lax.fori_loop(0, nsb, body, 0)
        if (nsb - 1) % 2 == 0:
            lgidx, ldst, lrows = gidxA, dstA, rowsA
        else:
            lgidx, ldst, lrows = gidxB, dstB, rowsB
        wait_gathers(lgidx, lrows)
        fire_scatters(ldst, lrows)
        wait_scatters(ldst, lrows)
        plsc.subcore_barrier()

        def copy_out(out_hbm):
            nchunks = out_rows // oc
            for k in range(nchunks):
                rbuf = rowsA if k % 2 == 0 else rowsB
                if k >= 2:
                    pltpu.make_async_copy(
                        rbuf.at[pl.ds(0, oc)],
                        out_hbm.at[pl.ds(sid * out_rows, oc)],
                        sem_o,
                    ).wait()
                pltpu.sync_copy(
                    acc_sh.at[pl.ds(sid * out_rows + k * oc, oc)],
                    rbuf.at[pl.ds(0, oc)],
                )
                pltpu.async_copy(
                    rbuf.at[pl.ds(0, oc)],
                    out_hbm.at[pl.ds(sid * out_rows + k * oc, oc)],
                    sem_o,
                )
            for k in range(min(2, nchunks)):
                rbuf = rowsA if (nchunks - 2 + k) % 2 == 0 else rowsB
                pltpu.make_async_copy(
                    rbuf.at[pl.ds(0, oc)],
                    out_hbm.at[pl.ds(sid * out_rows, oc)],
                    sem_o,
                ).wait()

        @pl.when(cid == 0)
        def _():
            copy_out(out_lo)

        @pl.when(cid == 1)
        def _():
            copy_out(out_hi)

    return edge_pass


def _full(rows, cols):
    return pl.BlockSpec((rows, cols), lambda i: (0, 0))


def _blk(rows, cols):
    return pl.BlockSpec((rows, cols), lambda i: (i, 0))


def _kdinv_body(d0_ref, d1_ref, b16_ref, b32_ref, b64_ref,
                r16_ref, r32_ref, r64_ref):
    dinv = lax.rsqrt(d0_ref[...] + d1_ref[...] + 1.0)
    r16_ref[...] = jnp.dot(dinv, b16_ref[...],
                           preferred_element_type=jnp.float32, precision=lax.Precision.HIGHEST)
    r32_ref[...] = jnp.dot(dinv, b32_ref[...],
                           preferred_element_type=jnp.float32, precision=lax.Precision.HIGHEST)
    r64_ref[...] = jnp.dot(dinv, b64_ref[...],
                           preferred_element_type=jnp.float32, precision=lax.Precision.HIGHEST)


def _ka_body(x_ref, w_ref, r32_ref, g0_ref):
    g0_ref[...] = r32_ref[...] * jnp.dot(
        x_ref[...], w_ref[...], preferred_element_type=jnp.float32, precision=lax.Precision.HIGHEST)


def _kb_body(alo_ref, ahi_ref, r16_ref, g0_ref, r32_ref, b_ref, w_ref,
             blo_ref, bhi_ref, h1_ref, g1_ref):
    r16 = r16_ref[...]
    accw = (jnp.dot(r16 * alo_ref[...], blo_ref[...],
                    preferred_element_type=jnp.float32, precision=lax.Precision.HIGHEST)
            + jnp.dot(r16 * ahi_ref[...], bhi_ref[...],
                      preferred_element_type=jnp.float32, precision=lax.Precision.HIGHEST))
    r32 = r32_ref[...]
    h1 = jnp.maximum(accw + r32 * g0_ref[...] + b_ref[...], 0.0)
    h1_ref[...] = h1
    g1_ref[...] = jnp.dot(r32 * h1, w_ref[...],
                          preferred_element_type=jnp.float32, precision=lax.Precision.HIGHEST)


def _kc_body(alo_ref, ahi_ref, r16_ref, g1_ref, r32_ref, h1_ref, b_ref,
             w_ref, blo_ref, bhi_ref, g2_ref):
    r16 = r16_ref[...]
    accw = (jnp.dot(r16 * alo_ref[...], blo_ref[...],
                    preferred_element_type=jnp.float32, precision=lax.Precision.HIGHEST)
            + jnp.dot(r16 * ahi_ref[...], bhi_ref[...],
                      preferred_element_type=jnp.float32, precision=lax.Precision.HIGHEST))
    r32 = r32_ref[...]
    h2 = (jnp.maximum(accw + r32 * g1_ref[...] + b_ref[...], 0.0)
          + h1_ref[...])
    g2_ref[...] = jnp.dot(r32 * h2, w_ref[...],
                          preferred_element_type=jnp.float32, precision=lax.Precision.HIGHEST)


def _kd_body(a0_ref, a1_ref, a2_ref, a3_ref, r16_ref, g2_ref, r64_ref,
             b_ref, p0_ref, p1_ref, p2_ref, p3_ref, out_ref):
    r16 = r16_ref[...]
    acc = jnp.dot(r16 * a0_ref[...], p0_ref[...],
                  preferred_element_type=jnp.float32, precision=lax.Precision.HIGHEST)
    acc = acc + jnp.dot(r16 * a1_ref[...], p1_ref[...],
                        preferred_element_type=jnp.float32, precision=lax.Precision.HIGHEST)
    acc = acc + jnp.dot(r16 * a2_ref[...], p2_ref[...],
                        preferred_element_type=jnp.float32, precision=lax.Precision.HIGHEST)
    acc = acc + jnp.dot(r16 * a3_ref[...], p3_ref[...],
                        preferred_element_type=jnp.float32, precision=lax.Precision.HIGHEST)
    out_ref[...] = acc + r64_ref[...] * g2_ref[...] + b_ref[...]


def kernel(x, edge_index, W0, b0, W1, b1, W_out, b_out):
    n, dfeat = x.shape
    e = edge_index.shape[1]
    nh = W0.shape[1]
    nclass = W_out.shape[1]
    grid_n = -(-n // _TC_R)
    n_sc = grid_n * _TC_R
    grid = (grid_n,)

    src = edge_index[0]
    dst = edge_index[1]
    e_pad = _round_up(e, NC * NS * SB)
    pad = e_pad - e
    src_p = jnp.concatenate([src, jnp.zeros((pad,), jnp.int32)]).reshape(-1, BATCH)
    dst_p = jnp.concatenate([dst, jnp.full((pad,), jnp.int32(n))]).reshape(-1, BATCH)

    m = np.arange(128)
    B16 = (m[:, None] == (np.arange(16 * 128) // 16)[None, :]).astype(np.float32)
    B32 = (m[:, None] == (np.arange(32 * 128) // 32)[None, :]).astype(np.float32)
    B64r = (m[:, None] == (np.arange(64 * 128) // 64)[None, :]).astype(np.float32)
    Blo = ((32 * (m // 16) + m % 16)[:, None]
           == np.arange(256)[None, :]).astype(np.float32)
    Bhi = ((32 * (m // 16) + 16 + m % 16)[:, None]
           == np.arange(256)[None, :]).astype(np.float32)
    B64 = [((64 * (m // 16) + 16 * j + m % 16)[:, None]
            == np.arange(512)[None, :]).astype(np.float32) for j in range(4)]

    W0bd = jnp.kron(jnp.eye(4, dtype=jnp.float32), W0)
    W1bd = jnp.kron(jnp.eye(8, dtype=jnp.float32), W1)
    Wobd = jnp.kron(jnp.eye(8, dtype=jnp.float32), W_out)
    b0w = jnp.tile(b0, 8)[None, :]
    b1w = jnp.tile(b1, 8)[None, :]
    bow = jnp.tile(b_out, 8)[None, :]

    deg_pp = _make_deg_kernel(n_sc, e_pad)(dst_p)
    d0 = deg_pp[:n_sc].reshape(n_sc // 128, 128)
    d1 = deg_pp[n_sc:].reshape(n_sc // 128, 128)

    pk1 = n_sc // 128
    pkf = n_sc * FH // 128

    kdinv = pl.pallas_call(
        _kdinv_body,
        grid=grid,
        in_specs=[_blk(16, 128), _blk(16, 128), _full(128, 2048),
                  _full(128, 4096), _full(128, 8192)],
        out_specs=[_blk(16, 2048), _blk(16, 4096), _blk(16, 8192)],
        out_shape=[
            jax.ShapeDtypeStruct((pk1, 2048), jnp.float32),
            jax.ShapeDtypeStruct((pk1, 4096), jnp.float32),
            jax.ShapeDtypeStruct((pk1, 8192), jnp.float32),
        ],
    )
    r16w, r32w, r64w = kdinv(d0, d1, B16, B32, B64r)
    rep16 = r16w.reshape(pkf, 128)
    rep32w = r32w.reshape(n_sc // 8, 256)
    rep32p = r32w.reshape(n_sc * 32 // 128, 128)
    rep64w = r64w.reshape(n_sc // 8, 512)

    ka = pl.pallas_call(
        _ka_body,
        grid=grid,
        in_specs=[_blk(512, 512), _full(512, 128), _blk(512, 128)],
        out_specs=_blk(512, 128),
        out_shape=jax.ShapeDtypeStruct((n_sc * 32 // 128, 128), jnp.float32),
    )
    g0p = ka(x.reshape(n // 4, 4 * dfeat), W0bd, rep32p)
    g0w = g0p.reshape(n_sc // 8, 256)

    ep2 = _make_edge_pass(n_sc, e_pad, 2, 0)
    a0lo, a0hi = ep2(g0p.reshape(2 * n_sc, FH), src_p, dst_p)

    kb = pl.pallas_call(
        _kb_body,
        grid=grid,
        in_specs=[_blk(256, 128), _blk(256, 128), _blk(256, 128),
                  _blk(256, 256), _blk(256, 256), _full(1, 256),
                  _full(256, 256), _full(128, 256), _full(128, 256)],
        out_specs=[_blk(256, 256), _blk(256, 256)],
        out_shape=[
            jax.ShapeDtypeStruct((n_sc // 8, 256), jnp.float32),
            jax.ShapeDtypeStruct((n_sc // 8, 256), jnp.float32),
        ],
    )
    h1w, g1w = kb(a0lo.reshape(pkf, 128), a0hi.reshape(pkf, 128), rep16,
                  g0w, rep32w, b0w, W1bd, Blo, Bhi)

    a1lo, a1hi = ep2(g1w.reshape(2 * n_sc, FH), src_p, dst_p)

    kc = pl.pallas_call(
        _kc_body,
        grid=grid,
        in_specs=[_blk(256, 128), _blk(256, 128), _blk(256, 128),
                  _blk(256, 256), _blk(256, 256), _blk(256, 256),
                  _full(1, 256), _full(256, 512), _full(128, 256),
                  _full(128, 256)],
        out_specs=_blk(256, 512),
        out_shape=jax.ShapeDtypeStruct((n_sc // 8, 512), jnp.float32),
    )
    g2w = kc(a1lo.reshape(pkf, 128), a1hi.reshape(pkf, 128), rep16,
             g1w, rep32w, h1w, b1w, Wobd, Blo, Bhi)

    g2_tbl = g2w.reshape(4 * n_sc, FH)
    ep4a = _make_edge_pass(n_sc, e_pad, 4, 0)
    ep4b = _make_edge_pass(n_sc, e_pad, 4, 2)
    a2q0, a2q1 = ep4a(g2_tbl, src_p, dst_p)
    a2q2, a2q3 = ep4b(g2_tbl, src_p, dst_p)

    kd = pl.pallas_call(
        _kd_body,
        grid=grid,
        in_specs=[_blk(256, 128)] * 4 + [_blk(256, 128), _blk(256, 512),
                  _blk(256, 512), _full(1, 512)]
                 + [_full(128, 512)] * 4,
        out_specs=_blk(256, 512),
        out_shape=jax.ShapeDtypeStruct((n_sc // 8, 512), jnp.float32),
    )
    outw = kd(a2q0.reshape(pkf, 128), a2q1.reshape(pkf, 128),
              a2q2.reshape(pkf, 128), a2q3.reshape(pkf, 128),
              rep16, g2w, rep64w, bow, B64[0], B64[1], B64[2], B64[3])
    return outw.reshape(n_sc, nclass)[:n]
